# baseline (speedup 1.0000x reference)
import functools
import jax
import jax.numpy as jnp
from jax import lax
from jax.experimental import pallas as pl
from jax.experimental.pallas import tpu as pltpu
from jax.experimental.pallas import tpu_sc as plsc

_F = 3 * 224 * 224
_CH = 37632
_NCHUNK = _F // _CH
_B = 64
_NW = 32
_BPW = _B // _NW


def _sc_body(x_hbm, t_hbm, n_hbm, sac_hbm, somac_hbm, out_hbm,
             sac_v, somac_v, t_v, x_v, n_v, o_v):
    wid = lax.axis_index("s") * 2 + lax.axis_index("c")
    pltpu.sync_copy(t_hbm, t_v)
    pltpu.sync_copy(sac_hbm, sac_v)
    pltpu.sync_copy(somac_hbm, somac_v)
    for bb in range(_BPW):
        b = wid * _BPW + bb
        b16 = jnp.zeros((16,), jnp.int32) + b
        t16 = plsc.load_gather(t_v, [b16])
        a16 = plsc.load_gather(sac_v, [t16])
        c16 = plsc.load_gather(somac_v, [t16])

        for k in range(_NCHUNK):
            base = b * _F + k * _CH
            pltpu.sync_copy(x_hbm.at[pl.ds(base, _CH)], x_v)
            pltpu.sync_copy(n_hbm.at[pl.ds(base, _CH)], n_v)

            @plsc.parallel_loop(0, _CH // 16, unroll=8)
            def _fma(i):
                s = pl.ds(i * 16, 16)
                o_v[s] = a16 * x_v[s] + c16 * n_v[s]

            pltpu.sync_copy(o_v, out_hbm.at[pl.ds(base, _CH)])


def kernel(x_0, t, noise, sqrt_alphas_cumprod, sqrt_one_minus_alphas_cumprod):
    xf = x_0.reshape(-1)
    nf = noise.reshape(-1)
    mesh = plsc.VectorSubcoreMesh(core_axis_name="c", subcore_axis_name="s")
    run = functools.partial(
        pl.kernel,
        mesh=mesh,
        compiler_params=pltpu.CompilerParams(needs_layout_passes=False),
        out_type=jax.ShapeDtypeStruct((_B * _F,), jnp.float32),
        scratch_types=[
            pltpu.VMEM((1000,), jnp.float32),
            pltpu.VMEM((1000,), jnp.float32),
            pltpu.VMEM((_B,), jnp.int32),
            pltpu.VMEM((_CH,), jnp.float32),
            pltpu.VMEM((_CH,), jnp.float32),
            pltpu.VMEM((_CH,), jnp.float32),
        ],
    )(_sc_body)
    noisy = run(xf, t, nf, sqrt_alphas_cumprod, sqrt_one_minus_alphas_cumprod)
    return noisy.reshape(x_0.shape), noise


# speedup vs baseline: 4.9144x; 4.9144x over previous
import jax
import jax.numpy as jnp
from jax.experimental import pallas as pl
from jax.experimental.pallas import tpu as pltpu

_C = 3
_H = 224
_W = 224
_BG = 8


def _fma_body(t_ref, sac_ref, somac_ref, x_ref, n_ref, out_ref, ncopy_ref):
    g = pl.program_id(0)
    for i in range(_BG):
        ti = t_ref[g * _BG + i]
        a = sac_ref[ti]
        c = somac_ref[ti]
        nv = n_ref[i]
        out_ref[i] = a * x_ref[i] + c * nv
        ncopy_ref[i] = nv


def kernel(x_0, t, noise, sqrt_alphas_cumprod, sqrt_one_minus_alphas_cumprod):
    batch = x_0.shape[0]

    grid_spec = pltpu.PrefetchScalarGridSpec(
        num_scalar_prefetch=3,
        grid=(batch // _BG,),
        in_specs=[
            pl.BlockSpec((_BG, _C, _H, _W), lambda g, t_r, sac_r, somac_r: (g, 0, 0, 0)),
            pl.BlockSpec((_BG, _C, _H, _W), lambda g, t_r, sac_r, somac_r: (g, 0, 0, 0)),
        ],
        out_specs=[
            pl.BlockSpec((_BG, _C, _H, _W), lambda g, t_r, sac_r, somac_r: (g, 0, 0, 0)),
            pl.BlockSpec((_BG, _C, _H, _W), lambda g, t_r, sac_r, somac_r: (g, 0, 0, 0)),
        ],
    )

    noisy, ncopy = pl.pallas_call(
        _fma_body,
        grid_spec=grid_spec,
        out_shape=[
            jax.ShapeDtypeStruct(x_0.shape, jnp.float32),
            jax.ShapeDtypeStruct(x_0.shape, jnp.float32),
        ],
    )(t, sqrt_alphas_cumprod, sqrt_one_minus_alphas_cumprod, x_0, noise)

    return noisy, ncopy
